# baseline (speedup 1.0000x reference)
import functools

import jax
import jax.numpy as jnp
from jax import lax
from jax.experimental import pallas as pl
from jax.experimental.pallas import tpu as pltpu
from jax.experimental.pallas import tpu_sc as plsc

B = 4
CAP = 256
IMG_LEN = 128 * 128
SEQ = CAP + 2 * IMG_LEN
N = B * SEQ
D0, D1, D2 = 32, 48, 48
DT = D0 + D1 + D2

NW = 32
PER_W = N // NW
M = 688
STEPS = PER_W // M


@functools.partial(
    pl.kernel,
    out_type=jax.ShapeDtypeStruct((N, DT), jnp.float32),
    mesh=plsc.VectorSubcoreMesh(core_axis_name="c", subcore_axis_name="s"),
    compiler_params=pltpu.CompilerParams(use_tc_tiling_on_sc=False),
    scratch_types=[
        pltpu.VMEM((M,), jnp.int32),
        pltpu.VMEM((M,), jnp.int32),
        pltpu.VMEM((M,), jnp.int32),
        pltpu.VMEM((M, D0), jnp.float32),
        pltpu.VMEM((M, D1), jnp.float32),
        pltpu.VMEM((M, D2), jnp.float32),
        pltpu.SemaphoreType.DMA,
    ],
)
def _rope_gather(t0, t1, t2, i0, i1, i2, out,
                 i0_v, i1_v, i2_v, r0_v, r1_v, r2_v, sem):
    nc = 2
    wid = lax.axis_index("s") * nc + lax.axis_index("c")
    wbase = wid * PER_W

    def body(c, carry):
        base = wbase + c * M
        pltpu.sync_copy(i0.at[pl.ds(base, M)], i0_v)
        pltpu.sync_copy(i1.at[pl.ds(base, M)], i1_v)
        pltpu.sync_copy(i2.at[pl.ds(base, M)], i2_v)
        cp0 = pltpu.async_copy(t0.at[i0_v], r0_v, sem)
        cp1 = pltpu.async_copy(t1.at[i1_v], r1_v, sem)
        cp2 = pltpu.async_copy(t2.at[i2_v], r2_v, sem)
        cp0.wait()
        cp1.wait()
        cp2.wait()
        pltpu.sync_copy(r0_v, out.at[pl.ds(base, M), pl.ds(0, D0)])
        pltpu.sync_copy(r1_v, out.at[pl.ds(base, M), pl.ds(D0, D1)])
        pltpu.sync_copy(r2_v, out.at[pl.ds(base, M), pl.ds(D0 + D1, D2)])
        return carry

    lax.fori_loop(0, STEPS, body, 0)


def kernel(encoder_hidden_states, freqs0, freqs1, freqs2, position_ids):
    del encoder_hidden_states
    t0 = freqs0.reshape(freqs0.shape[0], D0)
    t1 = freqs1.reshape(freqs1.shape[0], D1)
    t2 = freqs2.reshape(freqs2.shape[0], D2)
    pos = position_ids.reshape(N, 3).astype(jnp.int32)
    i0 = pos[:, 0]
    i1 = pos[:, 1]
    i2 = pos[:, 2]
    out = _rope_gather(t0, t1, t2, i0, i1, i2)
    return out.reshape(B, SEQ, DT // 2, 2)


# speedup vs baseline: 3.2061x; 3.2061x over previous
import functools

import jax
import jax.numpy as jnp
from jax import lax
from jax.experimental import pallas as pl
from jax.experimental.pallas import tpu as pltpu
from jax.experimental.pallas import tpu_sc as plsc

B = 4
CAP = 256
IMG_LEN = 128 * 128
SEQ = CAP + 2 * IMG_LEN
N = B * SEQ
D0, D1, D2 = 32, 48, 48
DT = D0 + D1 + D2
V0, V1, V2 = 1024, 512, 512

NW = 32
PER_W = N // NW
M = 688
STEPS = PER_W // M


@functools.partial(
    pl.kernel,
    out_type=jax.ShapeDtypeStruct((N, DT), jnp.float32),
    mesh=plsc.VectorSubcoreMesh(core_axis_name="c", subcore_axis_name="s"),
    compiler_params=pltpu.CompilerParams(use_tc_tiling_on_sc=False),
    scratch_types=[
        pltpu.VMEM_SHARED((V0, D0), jnp.float32),
        pltpu.VMEM_SHARED((V1, D1), jnp.float32),
        pltpu.VMEM_SHARED((V2, D2), jnp.float32),
        pltpu.VMEM((M,), jnp.int32),
        pltpu.VMEM((M,), jnp.int32),
        pltpu.VMEM((M,), jnp.int32),
        pltpu.VMEM((M, D0), jnp.float32),
        pltpu.VMEM((M, D1), jnp.float32),
        pltpu.VMEM((M, D2), jnp.float32),
        pltpu.SemaphoreType.DMA,
    ],
)
def _rope_gather(t0, t1, t2, i0, i1, i2, out,
                 t0_s, t1_s, t2_s, i0_v, i1_v, i2_v, r0_v, r1_v, r2_v, sem):
    nc = 2
    wid = lax.axis_index("s") * nc + lax.axis_index("c")
    wbase = wid * PER_W

    @pl.when(lax.axis_index("s") == 0)
    def _():
        pltpu.sync_copy(t0, t0_s)
        pltpu.sync_copy(t1, t1_s)
        pltpu.sync_copy(t2, t2_s)

    plsc.subcore_barrier()

    def body(c, carry):
        base = wbase + c * M
        pltpu.sync_copy(i0.at[pl.ds(base, M)], i0_v)
        pltpu.sync_copy(i1.at[pl.ds(base, M)], i1_v)
        pltpu.sync_copy(i2.at[pl.ds(base, M)], i2_v)
        cp0 = pltpu.async_copy(t0_s.at[i0_v], r0_v, sem)
        cp1 = pltpu.async_copy(t1_s.at[i1_v], r1_v, sem)
        cp2 = pltpu.async_copy(t2_s.at[i2_v], r2_v, sem)
        cp0.wait()
        cp1.wait()
        cp2.wait()
        pltpu.sync_copy(r0_v, out.at[pl.ds(base, M), pl.ds(0, D0)])
        pltpu.sync_copy(r1_v, out.at[pl.ds(base, M), pl.ds(D0, D1)])
        pltpu.sync_copy(r2_v, out.at[pl.ds(base, M), pl.ds(D0 + D1, D2)])
        return carry

    lax.fori_loop(0, STEPS, body, 0)


def kernel(encoder_hidden_states, freqs0, freqs1, freqs2, position_ids):
    del encoder_hidden_states
    t0 = freqs0.reshape(freqs0.shape[0], D0)
    t1 = freqs1.reshape(freqs1.shape[0], D1)
    t2 = freqs2.reshape(freqs2.shape[0], D2)
    pos = position_ids.reshape(N, 3).astype(jnp.int32)
    i0 = pos[:, 0]
    i1 = pos[:, 1]
    i2 = pos[:, 2]
    out = _rope_gather(t0, t1, t2, i0, i1, i2)
    return out.reshape(B, SEQ, DT // 2, 2)
